# Initial kernel scaffold; baseline (speedup 1.0000x reference)
import jax
import jax.numpy as jnp
from jax.experimental import pallas as pl


def kernel(x, edge_index, edge_attr, node_attr, additional_message_features, additional_node_features, W_emb, b_emb, Wm1_0, bm1_0, Wm2_0, bm2_0, Wu1_0, bu1_0, Wu2_0, bu2_0, Wm1_1, bm1_1, Wm2_1, bm2_1, Wu1_1, bu1_1, Wu2_1, bu2_1, Wp1, bp1, Wp2, bp2, Wp3, bp3):
    raise NotImplementedError("write your pallas kernel here")


# speedup vs baseline: 3.2530x; 3.2530x over previous
import functools
import math

import jax
import jax.numpy as jnp
from jax import lax
from jax.experimental import pallas as pl
from jax.experimental.pallas import tpu as pltpu
from jax.experimental.pallas import tpu_sc as plsc

_NC = 2
_NS = 16
_NW = _NC * _NS
_K = 80
_GRP = 8

_RE = 1280
_RN = 1000


def _silu(v):
    return v * lax.logistic(v)


def _sc_gather(ad, as_, dst3, src3):
    n, d = ad.shape
    ch = dst3.shape[1]
    ew = ch * _K
    e = _NW * ew
    mesh = plsc.VectorSubcoreMesh(core_axis_name="c", subcore_axis_name="s")
    out_t = jax.ShapeDtypeStruct((e, d), jnp.float32)

    @functools.partial(
        pl.kernel,
        out_type=(out_t, out_t),
        mesh=mesh,
        scratch_types=[
            pltpu.VMEM((ch, _K), jnp.int32),
            pltpu.VMEM((ch, _K), jnp.int32),
            pltpu.VMEM((_GRP, _K, d), jnp.float32),
            pltpu.SemaphoreType.DMA,
            pltpu.SemaphoreType.DMA,
        ],
    )
    def gather_kernel(ad_hbm, as_hbm, dst_hbm, src_hbm, td_hbm, ts_hbm,
                      idxd_v, idxs_v, bufs, gsem, wsem):
        wid = lax.axis_index("s") * _NC + lax.axis_index("c")
        base = wid * ew
        pltpu.sync_copy(dst_hbm.at[wid], idxd_v)
        pltpu.sync_copy(src_hbm.at[wid], idxs_v)

        for table, idx_v, out in ((ad_hbm, idxd_v, td_hbm),
                                  (as_hbm, idxs_v, ts_hbm)):
            def group(g, nb, table=table, idx_v=idx_v, out=out):
                cps = [pltpu.async_copy(table.at[idx_v.at[g + b]],
                                        bufs.at[b], gsem)
                       for b in range(nb)]
                for cp in cps:
                    cp.wait()
                cps = [pltpu.async_copy(bufs.at[b],
                                        out.at[pl.ds(base + (g + b) * _K, _K)],
                                        wsem)
                       for b in range(nb)]
                for cp in cps:
                    cp.wait()

            nfull, rem = ch // _GRP, ch % _GRP

            @pl.loop(0, nfull)
            def _(i):
                group(i * _GRP, _GRP)

            if rem:
                group(nfull * _GRP, rem)

    return gather_kernel(ad, as_, dst3, src3)


def _sc_scatter(m2, dst3, zeros):
    e, d = m2.shape
    n = zeros.shape[0]
    ch = dst3.shape[1]
    ew = ch * _K
    nr = (n // _NS) & ~7
    tail = n - nr * _NS
    mesh = plsc.VectorSubcoreMesh(core_axis_name="c", subcore_axis_name="s")

    grp = 3

    @functools.partial(
        pl.kernel,
        out_type=jax.ShapeDtypeStruct((_NC, n, d), jnp.float32),
        mesh=mesh,
        scratch_types=[
            pltpu.VMEM_SHARED((n, d), jnp.float32),
            pltpu.VMEM((ch, _K), jnp.int32),
            pltpu.VMEM((grp, _K, d), jnp.float32),
            pltpu.SemaphoreType.DMA,
            pltpu.SemaphoreType.DMA,
        ],
    )
    def scatter_kernel(m2_hbm, dst_hbm, z_hbm, out_hbm,
                       agg_sh, idx_v, bufs, rsem, asem):
        cid = lax.axis_index("c")
        sid = lax.axis_index("s")
        wid = sid * _NC + cid
        base = wid * ew
        pltpu.sync_copy(z_hbm.at[pl.ds(sid * nr, nr)],
                        agg_sh.at[pl.ds(sid * nr, nr)])
        if tail:
            @pl.when(sid == 0)
            def _():
                pltpu.sync_copy(z_hbm.at[pl.ds(nr * _NS, tail)],
                                agg_sh.at[pl.ds(nr * _NS, tail)])
        pltpu.sync_copy(dst_hbm.at[wid], idx_v)
        plsc.subcore_barrier()

        def group(g, nb):
            cps = [pltpu.async_copy(m2_hbm.at[pl.ds(base + (g + b) * _K, _K)],
                                    bufs.at[b], rsem)
                   for b in range(nb)]
            for cp in cps:
                cp.wait()
            cps = [pltpu.async_copy(bufs.at[b], agg_sh.at[idx_v.at[g + b]],
                                    asem, add=True)
                   for b in range(nb)]
            for cp in cps:
                cp.wait()

        nfull, rem = ch // grp, ch % grp

        @pl.loop(0, nfull)
        def _(i):
            group(i * grp, grp)

        if rem:
            group(nfull * grp, rem)

        plsc.subcore_barrier()
        pltpu.sync_copy(agg_sh.at[pl.ds(sid * nr, nr)],
                        out_hbm.at[cid, pl.ds(sid * nr, nr)])
        if tail:
            @pl.when(sid == 0)
            def _():
                pltpu.sync_copy(agg_sh.at[pl.ds(nr * _NS, tail)],
                                out_hbm.at[cid, pl.ds(nr * _NS, tail)])

    return scatter_kernel(m2, dst3, zeros)


def _dot(a, b):
    return jnp.dot(a, b, preferred_element_type=jnp.float32)


def _edge_body(td_ref, ts_ref, amf_ref, ea_ref, w1a_ref, bm1_ref,
               w2_ref, bm2_ref, out_ref, *, s1, s2):
    amf = amf_ref[...]
    ea = ea_ref[...]
    pre = (td_ref[...] + ts_ref[...] + amf * w1a_ref[...]) * (ea * s1) \
        + bm1_ref[...]
    m1 = _silu(pre)
    pre2 = _dot(m1, w2_ref[...]) * (ea * s2) + bm2_ref[...]
    out_ref[...] = _silu(pre2)


def _edge_call(td, ts, amf, ea, w1a, bm1, w2, bm2, s1, s2):
    e, d = td.shape
    grid = e // _RE
    row = lambda i: (i, 0)
    zero = lambda i: (0, 0)
    return pl.pallas_call(
        functools.partial(_edge_body, s1=s1, s2=s2),
        grid=(grid,),
        in_specs=[
            pl.BlockSpec((_RE, d), row),
            pl.BlockSpec((_RE, d), row),
            pl.BlockSpec((_RE, 1), row),
            pl.BlockSpec((_RE, 1), row),
            pl.BlockSpec((1, d), zero),
            pl.BlockSpec((1, d), zero),
            pl.BlockSpec((d, d), zero),
            pl.BlockSpec((1, d), zero),
        ],
        out_specs=pl.BlockSpec((_RE, d), row),
        out_shape=jax.ShapeDtypeStruct((e, d), jnp.float32),
    )(td, ts, amf, ea, w1a, bm1, w2, bm2)


def _emb_body(x_ref, anf_ref, na_ref, wem_ref, wea_ref, be_ref,
              w1dm_ref, w1da_ref, w1sm_ref, w1sa_ref,
              h_ref, ad_ref, as_ref, *, se):
    anf = anf_ref[...]
    na = na_ref[...]
    h = (_dot(x_ref[...], wem_ref[...]) + anf * wea_ref[...]) * (na * se) \
        + be_ref[...]
    h_ref[...] = h
    ad_ref[...] = _dot(h, w1dm_ref[...]) + anf * w1da_ref[...]
    as_ref[...] = _dot(h, w1sm_ref[...]) + anf * w1sa_ref[...]


def _emb_call(x, anf, na, wem, wea, be, w1dm, w1da, w1sm, w1sa, se):
    n, d = x.shape
    grid = n // _RN
    row = lambda i: (i, 0)
    zero = lambda i: (0, 0)
    nd = jax.ShapeDtypeStruct((n, d), jnp.float32)
    return pl.pallas_call(
        functools.partial(_emb_body, se=se),
        grid=(grid,),
        in_specs=[
            pl.BlockSpec((_RN, d), row),
            pl.BlockSpec((_RN, 1), row),
            pl.BlockSpec((_RN, 1), row),
            pl.BlockSpec((d, d), zero),
            pl.BlockSpec((1, d), zero),
            pl.BlockSpec((1, d), zero),
            pl.BlockSpec((d, d), zero),
            pl.BlockSpec((1, d), zero),
            pl.BlockSpec((d, d), zero),
            pl.BlockSpec((1, d), zero),
        ],
        out_specs=[pl.BlockSpec((_RN, d), row)] * 3,
        out_shape=[nd, nd, nd],
    )(x, anf, na, wem, wea, be, w1dm, w1da, w1sm, w1sa)


def _update_common(h_ref, anf_ref, na_ref, p0_ref, p1_ref,
                   wu1m_ref, wu1a_ref, wu1g_ref, bu1_ref, wu2_ref, bu2_ref,
                   su1, su2):
    anf = anf_ref[...]
    na = na_ref[...]
    h = h_ref[...]
    agg = p0_ref[...] + p1_ref[...]
    pre = (_dot(h, wu1m_ref[...]) + anf * wu1a_ref[...]
           + _dot(agg, wu1g_ref[...])) * (na * su1) + bu1_ref[...]
    u = _silu(pre)
    u2 = _dot(u, wu2_ref[...]) * (na * su2) + bu2_ref[...]
    return h + u2, anf, na


def _update_prep_body(h_ref, anf_ref, na_ref, p0_ref, p1_ref,
                      wu1m_ref, wu1a_ref, wu1g_ref, bu1_ref, wu2_ref, bu2_ref,
                      w1dm_ref, w1da_ref, w1sm_ref, w1sa_ref,
                      hn_ref, ad_ref, as_ref, *, su1, su2):
    hn, anf, _ = _update_common(h_ref, anf_ref, na_ref, p0_ref, p1_ref,
                                wu1m_ref, wu1a_ref, wu1g_ref, bu1_ref,
                                wu2_ref, bu2_ref, su1, su2)
    hn_ref[...] = hn
    ad_ref[...] = _dot(hn, w1dm_ref[...]) + anf * w1da_ref[...]
    as_ref[...] = _dot(hn, w1sm_ref[...]) + anf * w1sa_ref[...]


def _update_post_body(h_ref, anf_ref, na_ref, p0_ref, p1_ref,
                      wu1m_ref, wu1a_ref, wu1g_ref, bu1_ref, wu2_ref, bu2_ref,
                      wp1_ref, bp1_ref, wp2_ref, bp2_ref, wp3_ref, bp3_ref,
                      out_ref, *, su1, su2, sp):
    hn, _, na = _update_common(h_ref, anf_ref, na_ref, p0_ref, p1_ref,
                               wu1m_ref, wu1a_ref, wu1g_ref, bu1_ref,
                               wu2_ref, bu2_ref, su1, su2)
    q1 = _silu(_dot(hn, wp1_ref[...]) * (na * sp) + bp1_ref[...])
    q2 = _dot(q1, wp2_ref[...]) * (na * sp) + bp2_ref[...]
    out_ref[...] = _dot(q2, wp3_ref[...]) * (na * sp) + bp3_ref[...]


def _update_call(body, extra_w, nouts, h, anf, na, p0, p1,
                 wu1m, wu1a, wu1g, bu1, wu2, bu2):
    n, d = h.shape
    grid = n // _RN
    row = lambda i: (i, 0)
    zero = lambda i: (0, 0)
    nd = jax.ShapeDtypeStruct((n, d), jnp.float32)
    extra_specs = []
    for w in extra_w:
        extra_specs.append(pl.BlockSpec(w.shape, zero))
    return pl.pallas_call(
        body,
        grid=(grid,),
        in_specs=[
            pl.BlockSpec((_RN, d), row),
            pl.BlockSpec((_RN, 1), row),
            pl.BlockSpec((_RN, 1), row),
            pl.BlockSpec((_RN, d), row),
            pl.BlockSpec((_RN, d), row),
            pl.BlockSpec((d, d), zero),
            pl.BlockSpec((1, d), zero),
            pl.BlockSpec((d, d), zero),
            pl.BlockSpec((1, d), zero),
            pl.BlockSpec((d, d), zero),
            pl.BlockSpec((1, d), zero),
        ] + extra_specs,
        out_specs=[pl.BlockSpec((_RN, d), row)] * nouts,
        out_shape=[nd] * nouts,
    )(h, anf, na, p0, p1, wu1m, wu1a, wu1g, bu1, wu2, bu2, *extra_w)


def kernel(x, edge_index, edge_attr, node_attr, additional_message_features,
           additional_node_features, W_emb, b_emb, Wm1_0, bm1_0, Wm2_0, bm2_0,
           Wu1_0, bu1_0, Wu2_0, bu2_0, Wm1_1, bm1_1, Wm2_1, bm2_1,
           Wu1_1, bu1_1, Wu2_1, bu2_1, Wp1, bp1, Wp2, bp2, Wp3, bp3):
    n, d = x.shape
    e = edge_index.shape[1]
    anf = additional_node_features
    amf = additional_message_features
    na = node_attr
    ea = edge_attr

    ch = e // (_NW * _K)
    src3 = edge_index[0].reshape(_NW, ch, _K)
    dst3 = edge_index[1].reshape(_NW, ch, _K)
    zeros = jnp.zeros((n, d), jnp.float32)

    def split_m1(W):
        w = W[:, 0, :]
        return w[:d], w[d:d + 1], w[d + 1:2 * d + 1], w[2 * d + 1:2 * d + 2], \
            w[2 * d + 2:2 * d + 3]

    def split_u1(W):
        w = W[:, 0, :]
        return w[:d], w[d:d + 1], w[d + 1:d + 1 + d]

    rb = lambda b: b.reshape(1, d)
    wem = W_emb[:d, 0, :]
    wea = W_emb[d:d + 1, 0, :]
    se = 1.0 / math.sqrt(W_emb.shape[0])
    s1 = 1.0 / math.sqrt(Wm1_0.shape[0])
    s2 = 1.0 / math.sqrt(Wm2_0.shape[0])
    su1 = 1.0 / math.sqrt(Wu1_0.shape[0])
    su2 = 1.0 / math.sqrt(Wu2_0.shape[0])
    sp = 1.0 / math.sqrt(Wp1.shape[0])

    w1dm_0, w1da_0, w1sm_0, w1sa_0, w1amf_0 = split_m1(Wm1_0)
    w1dm_1, w1da_1, w1sm_1, w1sa_1, w1amf_1 = split_m1(Wm1_1)
    wu1m_0, wu1a_0, wu1g_0 = split_u1(Wu1_0)
    wu1m_1, wu1a_1, wu1g_1 = split_u1(Wu1_1)

    h0, ad0, as0 = _emb_call(x, anf, na, wem, wea, rb(b_emb),
                             w1dm_0, w1da_0, w1sm_0, w1sa_0, se)

    td0, ts0 = _sc_gather(ad0, as0, dst3, src3)
    m2_0 = _edge_call(td0, ts0, amf, ea, w1amf_0, rb(bm1_0),
                      Wm2_0[:, 0, :], rb(bm2_0), s1, s2)
    part0 = _sc_scatter(m2_0, dst3, zeros)
    h1, ad1, as1 = _update_call(
        functools.partial(_update_prep_body, su1=su1, su2=su2),
        [w1dm_1, w1da_1, w1sm_1, w1sa_1], 3,
        h0, anf, na, part0[0], part0[1],
        wu1m_0, wu1a_0, wu1g_0, rb(bu1_0), Wu2_0[:, 0, :], rb(bu2_0))

    td1, ts1 = _sc_gather(ad1, as1, dst3, src3)
    m2_1 = _edge_call(td1, ts1, amf, ea, w1amf_1, rb(bm1_1),
                      Wm2_1[:, 0, :], rb(bm2_1), s1, s2)
    part1 = _sc_scatter(m2_1, dst3, zeros)
    out = _update_call(
        functools.partial(_update_post_body, su1=su1, su2=su2, sp=sp),
        [Wp1[:, 0, :], rb(bp1), Wp2[:, 0, :], rb(bp2), Wp3[:, 0, :], rb(bp3)],
        1,
        h1, anf, na, part1[0], part1[1],
        wu1m_1, wu1a_1, wu1g_1, rb(bu1_1), Wu2_1[:, 0, :], rb(bu2_1))

    return out[0]


# speedup vs baseline: 3.5734x; 1.0985x over previous
import functools
import math

import jax
import jax.numpy as jnp
from jax import lax
from jax.experimental import pallas as pl
from jax.experimental.pallas import tpu as pltpu
from jax.experimental.pallas import tpu_sc as plsc

_NC = 2
_NS = 16
_NW = _NC * _NS
_K = 80
_GRP = 8

_RE = 1280
_RN = 1000


def _silu(v):
    return v * lax.logistic(v)


def _sc_gather(ad, as_, dst3, src3):
    n, d = ad.shape
    ch = dst3.shape[1]
    ew = ch * _K
    e = _NW * ew
    mesh = plsc.VectorSubcoreMesh(core_axis_name="c", subcore_axis_name="s")
    out_t = jax.ShapeDtypeStruct((e, d), jnp.float32)

    @functools.partial(
        pl.kernel,
        out_type=(out_t, out_t),
        mesh=mesh,
        scratch_types=[
            pltpu.VMEM((ch, _K), jnp.int32),
            pltpu.VMEM((ch, _K), jnp.int32),
            pltpu.VMEM((_GRP, _K, d), jnp.float32),
            pltpu.SemaphoreType.DMA,
            pltpu.SemaphoreType.DMA,
        ],
    )
    def gather_kernel(ad_hbm, as_hbm, dst_hbm, src_hbm, td_hbm, ts_hbm,
                      idxd_v, idxs_v, bufs, gsem, wsem):
        wid = lax.axis_index("s") * _NC + lax.axis_index("c")
        base = wid * ew
        pltpu.sync_copy(dst_hbm.at[wid], idxd_v)
        pltpu.sync_copy(src_hbm.at[wid], idxs_v)

        for table, idx_v, out in ((ad_hbm, idxd_v, td_hbm),
                                  (as_hbm, idxs_v, ts_hbm)):
            def group(g, nb, table=table, idx_v=idx_v, out=out):
                cps = [pltpu.async_copy(table.at[idx_v.at[g + b]],
                                        bufs.at[b], gsem)
                       for b in range(nb)]
                for cp in cps:
                    cp.wait()
                cps = [pltpu.async_copy(bufs.at[b],
                                        out.at[pl.ds(base + (g + b) * _K, _K)],
                                        wsem)
                       for b in range(nb)]
                for cp in cps:
                    cp.wait()

            nfull, rem = ch // _GRP, ch % _GRP

            @pl.loop(0, nfull)
            def _(i):
                group(i * _GRP, _GRP)

            if rem:
                group(nfull * _GRP, rem)

    return gather_kernel(ad, as_, dst3, src3)


def _sc_scatter(m2, dst3, zeros):
    e, d = m2.shape
    n = zeros.shape[0]
    ch = dst3.shape[1]
    ew = ch * _K
    nr = (n // _NS) & ~7
    tail = n - nr * _NS
    mesh = plsc.VectorSubcoreMesh(core_axis_name="c", subcore_axis_name="s")

    grp = 3

    @functools.partial(
        pl.kernel,
        out_type=jax.ShapeDtypeStruct((_NC, n, d), jnp.float32),
        mesh=mesh,
        scratch_types=[
            pltpu.VMEM_SHARED((n, d), jnp.float32),
            pltpu.VMEM((ch, _K), jnp.int32),
            pltpu.VMEM((grp, _K, d), jnp.float32),
            pltpu.SemaphoreType.DMA,
            pltpu.SemaphoreType.DMA,
        ],
    )
    def scatter_kernel(m2_hbm, dst_hbm, z_hbm, out_hbm,
                       agg_sh, idx_v, bufs, rsem, asem):
        cid = lax.axis_index("c")
        sid = lax.axis_index("s")
        wid = sid * _NC + cid
        base = wid * ew
        pltpu.sync_copy(z_hbm.at[pl.ds(sid * nr, nr)],
                        agg_sh.at[pl.ds(sid * nr, nr)])
        if tail:
            @pl.when(sid == 0)
            def _():
                pltpu.sync_copy(z_hbm.at[pl.ds(nr * _NS, tail)],
                                agg_sh.at[pl.ds(nr * _NS, tail)])
        pltpu.sync_copy(dst_hbm.at[wid], idx_v)
        plsc.subcore_barrier()

        def group(g, nb):
            cps = [pltpu.async_copy(m2_hbm.at[pl.ds(base + (g + b) * _K, _K)],
                                    bufs.at[b], rsem)
                   for b in range(nb)]
            for cp in cps:
                cp.wait()
            cps = [pltpu.async_copy(bufs.at[b], agg_sh.at[idx_v.at[g + b]],
                                    asem, add=True)
                   for b in range(nb)]
            for cp in cps:
                cp.wait()

        nfull, rem = ch // grp, ch % grp

        @pl.loop(0, nfull)
        def _(i):
            group(i * grp, grp)

        if rem:
            group(nfull * grp, rem)

        plsc.subcore_barrier()
        pltpu.sync_copy(agg_sh.at[pl.ds(sid * nr, nr)],
                        out_hbm.at[cid, pl.ds(sid * nr, nr)])
        if tail:
            @pl.when(sid == 0)
            def _():
                pltpu.sync_copy(agg_sh.at[pl.ds(nr * _NS, tail)],
                                out_hbm.at[cid, pl.ds(nr * _NS, tail)])

    return scatter_kernel(m2, dst3, zeros)


def _dot(a, b):
    return jnp.dot(a, b, preferred_element_type=jnp.float32)


def _edge_body(td_ref, ts_ref, amf_ref, ea_ref, w1a_ref, bm1_ref,
               w2_ref, bm2_ref, out_ref, *, s1, s2):
    amf = amf_ref[...]
    ea = ea_ref[...]
    pre = (td_ref[...] + ts_ref[...] + amf * w1a_ref[...]) * (ea * s1) \
        + bm1_ref[...]
    m1 = _silu(pre)
    pre2 = _dot(m1, w2_ref[...]) * (ea * s2) + bm2_ref[...]
    out_ref[...] = _silu(pre2)


def _edge_call(td, ts, amf, ea, w1a, bm1, w2, bm2, s1, s2):
    e, d = td.shape
    grid = e // _RE
    row = lambda i: (i, 0)
    zero = lambda i: (0, 0)
    return pl.pallas_call(
        functools.partial(_edge_body, s1=s1, s2=s2),
        grid=(grid,),
        in_specs=[
            pl.BlockSpec((_RE, d), row),
            pl.BlockSpec((_RE, d), row),
            pl.BlockSpec((_RE, 1), row),
            pl.BlockSpec((_RE, 1), row),
            pl.BlockSpec((1, d), zero),
            pl.BlockSpec((1, d), zero),
            pl.BlockSpec((d, d), zero),
            pl.BlockSpec((1, d), zero),
        ],
        out_specs=pl.BlockSpec((_RE, d), row),
        out_shape=jax.ShapeDtypeStruct((e, d), jnp.float32),
    )(td, ts, amf, ea, w1a, bm1, w2, bm2)


def _emb_body(x_ref, anf_ref, na_ref, wem_ref, wea_ref, be_ref,
              w1dm_ref, w1da_ref, w1sm_ref, w1sa_ref,
              h_ref, ad_ref, as_ref, *, se):
    anf = anf_ref[...]
    na = na_ref[...]
    h = (_dot(x_ref[...], wem_ref[...]) + anf * wea_ref[...]) * (na * se) \
        + be_ref[...]
    h_ref[...] = h
    ad_ref[...] = _dot(h, w1dm_ref[...]) + anf * w1da_ref[...]
    as_ref[...] = _dot(h, w1sm_ref[...]) + anf * w1sa_ref[...]


def _emb_call(x, anf, na, wem, wea, be, w1dm, w1da, w1sm, w1sa, se):
    n, d = x.shape
    grid = n // _RN
    row = lambda i: (i, 0)
    zero = lambda i: (0, 0)
    nd = jax.ShapeDtypeStruct((n, d), jnp.float32)
    return pl.pallas_call(
        functools.partial(_emb_body, se=se),
        grid=(grid,),
        in_specs=[
            pl.BlockSpec((_RN, d), row),
            pl.BlockSpec((_RN, 1), row),
            pl.BlockSpec((_RN, 1), row),
            pl.BlockSpec((d, d), zero),
            pl.BlockSpec((1, d), zero),
            pl.BlockSpec((1, d), zero),
            pl.BlockSpec((d, d), zero),
            pl.BlockSpec((1, d), zero),
            pl.BlockSpec((d, d), zero),
            pl.BlockSpec((1, d), zero),
        ],
        out_specs=[pl.BlockSpec((_RN, d), row)] * 3,
        out_shape=[nd, nd, nd],
    )(x, anf, na, wem, wea, be, w1dm, w1da, w1sm, w1sa)


def _update_common(h_ref, anf_ref, na_ref, parts,
                   wu1m_ref, wu1a_ref, wu1g_ref, bu1_ref, wu2_ref, bu2_ref,
                   su1, su2):
    anf = anf_ref[...]
    na = na_ref[...]
    h = h_ref[...]
    agg = parts[0][...]
    for p in parts[1:]:
        agg = agg + p[...]
    pre = (_dot(h, wu1m_ref[...]) + anf * wu1a_ref[...]
           + _dot(agg, wu1g_ref[...])) * (na * su1) + bu1_ref[...]
    u = _silu(pre)
    u2 = _dot(u, wu2_ref[...]) * (na * su2) + bu2_ref[...]
    return h + u2, anf, na


def _update_prep_body(h_ref, anf_ref, na_ref, *rest, nparts, su1, su2):
    parts = rest[:nparts]
    (wu1m_ref, wu1a_ref, wu1g_ref, bu1_ref, wu2_ref, bu2_ref,
     w1dm_ref, w1da_ref, w1sm_ref, w1sa_ref,
     hn_ref, ad_ref, as_ref) = rest[nparts:]
    hn, anf, _ = _update_common(h_ref, anf_ref, na_ref, parts,
                                wu1m_ref, wu1a_ref, wu1g_ref, bu1_ref,
                                wu2_ref, bu2_ref, su1, su2)
    hn_ref[...] = hn
    ad_ref[...] = _dot(hn, w1dm_ref[...]) + anf * w1da_ref[...]
    as_ref[...] = _dot(hn, w1sm_ref[...]) + anf * w1sa_ref[...]


def _update_post_body(h_ref, anf_ref, na_ref, *rest, nparts, su1, su2, sp):
    parts = rest[:nparts]
    (wu1m_ref, wu1a_ref, wu1g_ref, bu1_ref, wu2_ref, bu2_ref,
     wp1_ref, bp1_ref, wp2_ref, bp2_ref, wp3_ref, bp3_ref,
     out_ref) = rest[nparts:]
    hn, _, na = _update_common(h_ref, anf_ref, na_ref, parts,
                               wu1m_ref, wu1a_ref, wu1g_ref, bu1_ref,
                               wu2_ref, bu2_ref, su1, su2)
    q1 = _silu(_dot(hn, wp1_ref[...]) * (na * sp) + bp1_ref[...])
    q2 = _dot(q1, wp2_ref[...]) * (na * sp) + bp2_ref[...]
    out_ref[...] = _dot(q2, wp3_ref[...]) * (na * sp) + bp3_ref[...]


def _update_call(body, extra_w, nouts, h, anf, na, parts,
                 wu1m, wu1a, wu1g, bu1, wu2, bu2):
    n, d = h.shape
    grid = n // _RN
    row = lambda i: (i, 0)
    zero = lambda i: (0, 0)
    nd = jax.ShapeDtypeStruct((n, d), jnp.float32)
    extra_specs = []
    for w in extra_w:
        extra_specs.append(pl.BlockSpec(w.shape, zero))
    return pl.pallas_call(
        body,
        grid=(grid,),
        in_specs=[
            pl.BlockSpec((_RN, d), row),
            pl.BlockSpec((_RN, 1), row),
            pl.BlockSpec((_RN, 1), row),
        ] + [pl.BlockSpec((_RN, d), row)] * len(parts) + [
            pl.BlockSpec((d, d), zero),
            pl.BlockSpec((1, d), zero),
            pl.BlockSpec((d, d), zero),
            pl.BlockSpec((1, d), zero),
            pl.BlockSpec((d, d), zero),
            pl.BlockSpec((1, d), zero),
        ] + extra_specs,
        out_specs=[pl.BlockSpec((_RN, d), row)] * nouts,
        out_shape=[nd] * nouts,
    )(h, anf, na, *parts, wu1m, wu1a, wu1g, bu1, wu2, bu2, *extra_w)


def kernel(x, edge_index, edge_attr, node_attr, additional_message_features,
           additional_node_features, W_emb, b_emb, Wm1_0, bm1_0, Wm2_0, bm2_0,
           Wu1_0, bu1_0, Wu2_0, bu2_0, Wm1_1, bm1_1, Wm2_1, bm2_1,
           Wu1_1, bu1_1, Wu2_1, bu2_1, Wp1, bp1, Wp2, bp2, Wp3, bp3):
    n, d = x.shape
    e = edge_index.shape[1]
    anf = additional_node_features
    amf = additional_message_features
    na = node_attr
    ea = edge_attr

    nch = 5
    ec = e // nch
    ch = ec // (_NW * _K)
    src4 = edge_index[0].reshape(nch, _NW, ch, _K)
    dst4 = edge_index[1].reshape(nch, _NW, ch, _K)
    amf4 = amf.reshape(nch, ec, 1)
    ea4 = ea.reshape(nch, ec, 1)
    zeros = jnp.zeros((n, d), jnp.float32)

    def split_m1(W):
        w = W[:, 0, :]
        return w[:d], w[d:d + 1], w[d + 1:2 * d + 1], w[2 * d + 1:2 * d + 2], \
            w[2 * d + 2:2 * d + 3]

    def split_u1(W):
        w = W[:, 0, :]
        return w[:d], w[d:d + 1], w[d + 1:d + 1 + d]

    rb = lambda b: b.reshape(1, d)
    wem = W_emb[:d, 0, :]
    wea = W_emb[d:d + 1, 0, :]
    se = 1.0 / math.sqrt(W_emb.shape[0])
    s1 = 1.0 / math.sqrt(Wm1_0.shape[0])
    s2 = 1.0 / math.sqrt(Wm2_0.shape[0])
    su1 = 1.0 / math.sqrt(Wu1_0.shape[0])
    su2 = 1.0 / math.sqrt(Wu2_0.shape[0])
    sp = 1.0 / math.sqrt(Wp1.shape[0])

    w1dm_0, w1da_0, w1sm_0, w1sa_0, w1amf_0 = split_m1(Wm1_0)
    w1dm_1, w1da_1, w1sm_1, w1sa_1, w1amf_1 = split_m1(Wm1_1)
    wu1m_0, wu1a_0, wu1g_0 = split_u1(Wu1_0)
    wu1m_1, wu1a_1, wu1g_1 = split_u1(Wu1_1)

    h0, ad0, as0 = _emb_call(x, anf, na, wem, wea, rb(b_emb),
                             w1dm_0, w1da_0, w1sm_0, w1sa_0, se)

    def layer(ad, as_, w1amf, bm1, wm2, bm2):
        parts = []
        for c in range(nch):
            td, ts = _sc_gather(ad, as_, dst4[c], src4[c])
            m2 = _edge_call(td, ts, amf4[c], ea4[c], w1amf, rb(bm1),
                            wm2[:, 0, :], rb(bm2), s1, s2)
            p = _sc_scatter(m2, dst4[c], zeros)
            parts += [p[0], p[1]]
        return parts

    parts0 = layer(ad0, as0, w1amf_0, bm1_0, Wm2_0, bm2_0)
    h1, ad1, as1 = _update_call(
        functools.partial(_update_prep_body, nparts=len(parts0),
                          su1=su1, su2=su2),
        [w1dm_1, w1da_1, w1sm_1, w1sa_1], 3,
        h0, anf, na, parts0,
        wu1m_0, wu1a_0, wu1g_0, rb(bu1_0), Wu2_0[:, 0, :], rb(bu2_0))

    parts1 = layer(ad1, as1, w1amf_1, bm1_1, Wm2_1, bm2_1)
    out = _update_call(
        functools.partial(_update_post_body, nparts=len(parts1),
                          su1=su1, su2=su2, sp=sp),
        [Wp1[:, 0, :], rb(bp1), Wp2[:, 0, :], rb(bp2), Wp3[:, 0, :], rb(bp3)],
        1,
        h1, anf, na, parts1,
        wu1m_1, wu1a_1, wu1g_1, rb(bu1_1), Wu2_1[:, 0, :], rb(bu2_1))

    return out[0]


# speedup vs baseline: 4.2321x; 1.1843x over previous
import functools
import math

import jax
import jax.numpy as jnp
from jax import lax
from jax.experimental import pallas as pl
from jax.experimental.pallas import tpu as pltpu
from jax.experimental.pallas import tpu_sc as plsc

_NC = 2
_NS = 16
_NW = _NC * _NS
_K = 80
_GRP = 8

_RE = 1280
_RN = 1000


def _silu(v):
    return v * lax.logistic(v)


def _sc_gather(ad, as_, dst3, src3):
    n, d = ad.shape
    ch = dst3.shape[1]
    ew = ch * _K
    e = _NW * ew
    mesh = plsc.VectorSubcoreMesh(core_axis_name="c", subcore_axis_name="s")
    out_t = jax.ShapeDtypeStruct((e, d), jnp.float32)

    @functools.partial(
        pl.kernel,
        out_type=(out_t, out_t),
        mesh=mesh,
        scratch_types=[
            pltpu.VMEM((ch, _K), jnp.int32),
            pltpu.VMEM((ch, _K), jnp.int32),
            pltpu.VMEM((_GRP, _K, d), jnp.float32),
            pltpu.SemaphoreType.DMA,
            pltpu.SemaphoreType.DMA,
        ],
    )
    def gather_kernel(ad_hbm, as_hbm, dst_hbm, src_hbm, td_hbm, ts_hbm,
                      idxd_v, idxs_v, bufs, gsem, wsem):
        wid = lax.axis_index("s") * _NC + lax.axis_index("c")
        base = wid * ew
        pltpu.sync_copy(dst_hbm.at[wid], idxd_v)
        pltpu.sync_copy(src_hbm.at[wid], idxs_v)

        for table, idx_v, out in ((ad_hbm, idxd_v, td_hbm),
                                  (as_hbm, idxs_v, ts_hbm)):
            def group(g, nb, table=table, idx_v=idx_v, out=out):
                cps = [pltpu.async_copy(table.at[idx_v.at[g + b]],
                                        bufs.at[b], gsem)
                       for b in range(nb)]
                for cp in cps:
                    cp.wait()
                cps = [pltpu.async_copy(bufs.at[b],
                                        out.at[pl.ds(base + (g + b) * _K, _K)],
                                        wsem)
                       for b in range(nb)]
                for cp in cps:
                    cp.wait()

            nfull, rem = ch // _GRP, ch % _GRP

            @pl.loop(0, nfull)
            def _(i):
                group(i * _GRP, _GRP)

            if rem:
                group(nfull * _GRP, rem)

    return gather_kernel(ad, as_, dst3, src3)


def _sc_scatter(m2, dst3, zeros):
    e, d = m2.shape
    n = zeros.shape[0]
    ch = dst3.shape[1]
    ew = ch * _K
    nr = (n // _NS) & ~7
    tail = n - nr * _NS
    mesh = plsc.VectorSubcoreMesh(core_axis_name="c", subcore_axis_name="s")

    grp = 3

    @functools.partial(
        pl.kernel,
        out_type=jax.ShapeDtypeStruct((_NC, n, d), jnp.float32),
        mesh=mesh,
        scratch_types=[
            pltpu.VMEM_SHARED((n, d), jnp.float32),
            pltpu.VMEM((ch, _K), jnp.int32),
            pltpu.VMEM((grp, _K, d), jnp.float32),
            pltpu.SemaphoreType.DMA,
            pltpu.SemaphoreType.DMA,
        ],
    )
    def scatter_kernel(m2_hbm, dst_hbm, z_hbm, out_hbm,
                       agg_sh, idx_v, bufs, rsem, asem):
        cid = lax.axis_index("c")
        sid = lax.axis_index("s")
        wid = sid * _NC + cid
        base = wid * ew
        pltpu.sync_copy(z_hbm.at[pl.ds(sid * nr, nr)],
                        agg_sh.at[pl.ds(sid * nr, nr)])
        if tail:
            @pl.when(sid == 0)
            def _():
                pltpu.sync_copy(z_hbm.at[pl.ds(nr * _NS, tail)],
                                agg_sh.at[pl.ds(nr * _NS, tail)])
        pltpu.sync_copy(dst_hbm.at[wid], idx_v)
        plsc.subcore_barrier()

        def group(g, nb):
            cps = [pltpu.async_copy(m2_hbm.at[pl.ds(base + (g + b) * _K, _K)],
                                    bufs.at[b], rsem)
                   for b in range(nb)]
            for cp in cps:
                cp.wait()
            cps = [pltpu.async_copy(bufs.at[b], agg_sh.at[idx_v.at[g + b]],
                                    asem, add=True)
                   for b in range(nb)]
            for cp in cps:
                cp.wait()

        nfull, rem = ch // grp, ch % grp

        @pl.loop(0, nfull)
        def _(i):
            group(i * grp, grp)

        if rem:
            group(nfull * grp, rem)

        plsc.subcore_barrier()
        pltpu.sync_copy(agg_sh.at[pl.ds(sid * nr, nr)],
                        out_hbm.at[cid, pl.ds(sid * nr, nr)])
        if tail:
            @pl.when(sid == 0)
            def _():
                pltpu.sync_copy(agg_sh.at[pl.ds(nr * _NS, tail)],
                                out_hbm.at[cid, pl.ds(nr * _NS, tail)])

    return scatter_kernel(m2, dst3, zeros)


def _dot(a, b):
    return jnp.dot(a, b, preferred_element_type=jnp.float32)


def _expand_rows(v, d):
    g = v.shape[0]
    ones = jnp.ones((1, d), jnp.float32)
    cols = [lax.dot_general(v[i:i + 1, :], ones, (((0,), (0,)), ((), ())),
                            preferred_element_type=jnp.float32)
            for i in range(g)]
    return jnp.concatenate(cols, axis=0)


def _edge_body(td_ref, ts_ref, amf_ref, ea_ref, w1a_ref, bm1_ref,
               w2_ref, bm2_ref, out_ref, *, s1, s2):
    d = td_ref.shape[1]
    gr = amf_ref.shape[0]
    amf = _expand_rows(amf_ref[...].reshape(gr, 128), d)
    ea = _expand_rows(ea_ref[...].reshape(gr, 128), d)
    t = td_ref[...] + ts_ref[...]
    pre = (t + amf * w1a_ref[...]) * (ea * s1) + bm1_ref[...]
    m1 = _silu(pre)
    pre2 = _dot(m1, w2_ref[...]) * (ea * s2) + bm2_ref[...]
    out_ref[...] = _silu(pre2)


def _edge_call(td, ts, amf_l, ea_l, w1a, bm1, w2, bm2, s1, s2):
    e, d = td.shape
    grid = e // _RE
    gr = _RE // 128
    row = lambda i: (i, 0)
    zero = lambda i: (0, 0)
    return pl.pallas_call(
        functools.partial(_edge_body, s1=s1, s2=s2),
        grid=(grid,),
        in_specs=[
            pl.BlockSpec((_RE, d), row),
            pl.BlockSpec((_RE, d), row),
            pl.BlockSpec((gr, 1, 128), lambda i: (i, 0, 0)),
            pl.BlockSpec((gr, 1, 128), lambda i: (i, 0, 0)),
            pl.BlockSpec((1, d), zero),
            pl.BlockSpec((1, d), zero),
            pl.BlockSpec((d, d), zero),
            pl.BlockSpec((1, d), zero),
        ],
        out_specs=pl.BlockSpec((_RE, d), row),
        out_shape=jax.ShapeDtypeStruct((e, d), jnp.float32),
    )(td, ts, amf_l, ea_l, w1a, bm1, w2, bm2)


def _emb_body(x_ref, anf_ref, na_ref, wem_ref, wea_ref, be_ref,
              w1dm_ref, w1da_ref, w1sm_ref, w1sa_ref,
              h_ref, ad_ref, as_ref, *, se):
    anf = anf_ref[...]
    na = na_ref[...]
    h = (_dot(x_ref[...], wem_ref[...]) + anf * wea_ref[...]) * (na * se) \
        + be_ref[...]
    h_ref[...] = h
    ad_ref[...] = _dot(h, w1dm_ref[...]) + anf * w1da_ref[...]
    as_ref[...] = _dot(h, w1sm_ref[...]) + anf * w1sa_ref[...]


def _emb_call(x, anf, na, wem, wea, be, w1dm, w1da, w1sm, w1sa, se):
    n, d = x.shape
    grid = n // _RN
    row = lambda i: (i, 0)
    zero = lambda i: (0, 0)
    nd = jax.ShapeDtypeStruct((n, d), jnp.float32)
    return pl.pallas_call(
        functools.partial(_emb_body, se=se),
        grid=(grid,),
        in_specs=[
            pl.BlockSpec((_RN, d), row),
            pl.BlockSpec((_RN, 1), row),
            pl.BlockSpec((_RN, 1), row),
            pl.BlockSpec((d, d), zero),
            pl.BlockSpec((1, d), zero),
            pl.BlockSpec((1, d), zero),
            pl.BlockSpec((d, d), zero),
            pl.BlockSpec((1, d), zero),
            pl.BlockSpec((d, d), zero),
            pl.BlockSpec((1, d), zero),
        ],
        out_specs=[pl.BlockSpec((_RN, d), row)] * 3,
        out_shape=[nd, nd, nd],
    )(x, anf, na, wem, wea, be, w1dm, w1da, w1sm, w1sa)


def _update_common(h_ref, anf_ref, na_ref, parts,
                   wu1m_ref, wu1a_ref, wu1g_ref, bu1_ref, wu2_ref, bu2_ref,
                   su1, su2):
    anf = anf_ref[...]
    na = na_ref[...]
    h = h_ref[...]
    agg = parts[0][...]
    for p in parts[1:]:
        agg = agg + p[...]
    pre = (_dot(h, wu1m_ref[...]) + anf * wu1a_ref[...]
           + _dot(agg, wu1g_ref[...])) * (na * su1) + bu1_ref[...]
    u = _silu(pre)
    u2 = _dot(u, wu2_ref[...]) * (na * su2) + bu2_ref[...]
    return h + u2, anf, na


def _update_prep_body(h_ref, anf_ref, na_ref, *rest, nparts, su1, su2):
    parts = rest[:nparts]
    (wu1m_ref, wu1a_ref, wu1g_ref, bu1_ref, wu2_ref, bu2_ref,
     w1dm_ref, w1da_ref, w1sm_ref, w1sa_ref,
     hn_ref, ad_ref, as_ref) = rest[nparts:]
    hn, anf, _ = _update_common(h_ref, anf_ref, na_ref, parts,
                                wu1m_ref, wu1a_ref, wu1g_ref, bu1_ref,
                                wu2_ref, bu2_ref, su1, su2)
    hn_ref[...] = hn
    ad_ref[...] = _dot(hn, w1dm_ref[...]) + anf * w1da_ref[...]
    as_ref[...] = _dot(hn, w1sm_ref[...]) + anf * w1sa_ref[...]


def _update_post_body(h_ref, anf_ref, na_ref, *rest, nparts, su1, su2, sp):
    parts = rest[:nparts]
    (wu1m_ref, wu1a_ref, wu1g_ref, bu1_ref, wu2_ref, bu2_ref,
     wp1_ref, bp1_ref, wp2_ref, bp2_ref, wp3_ref, bp3_ref,
     out_ref) = rest[nparts:]
    hn, _, na = _update_common(h_ref, anf_ref, na_ref, parts,
                               wu1m_ref, wu1a_ref, wu1g_ref, bu1_ref,
                               wu2_ref, bu2_ref, su1, su2)
    q1 = _silu(_dot(hn, wp1_ref[...]) * (na * sp) + bp1_ref[...])
    q2 = _dot(q1, wp2_ref[...]) * (na * sp) + bp2_ref[...]
    out_ref[...] = _dot(q2, wp3_ref[...]) * (na * sp) + bp3_ref[...]


def _update_call(body, extra_w, out_widths, h, anf, na, parts,
                 wu1m, wu1a, wu1g, bu1, wu2, bu2):
    n, d = h.shape
    grid = n // _RN
    row = lambda i: (i, 0)
    zero = lambda i: (0, 0)
    extra_specs = []
    for w in extra_w:
        extra_specs.append(pl.BlockSpec(w.shape, zero))
    return pl.pallas_call(
        body,
        grid=(grid,),
        in_specs=[
            pl.BlockSpec((_RN, d), row),
            pl.BlockSpec((_RN, 1), row),
            pl.BlockSpec((_RN, 1), row),
        ] + [pl.BlockSpec((_RN, d), row)] * len(parts) + [
            pl.BlockSpec((d, d), zero),
            pl.BlockSpec((1, d), zero),
            pl.BlockSpec((d, d), zero),
            pl.BlockSpec((1, d), zero),
            pl.BlockSpec((d, d), zero),
            pl.BlockSpec((1, d), zero),
        ] + extra_specs,
        out_specs=[pl.BlockSpec((_RN, w), row) for w in out_widths],
        out_shape=[jax.ShapeDtypeStruct((n, w), jnp.float32)
                   for w in out_widths],
    )(h, anf, na, *parts, wu1m, wu1a, wu1g, bu1, wu2, bu2, *extra_w)


def kernel(x, edge_index, edge_attr, node_attr, additional_message_features,
           additional_node_features, W_emb, b_emb, Wm1_0, bm1_0, Wm2_0, bm2_0,
           Wu1_0, bu1_0, Wu2_0, bu2_0, Wm1_1, bm1_1, Wm2_1, bm2_1,
           Wu1_1, bu1_1, Wu2_1, bu2_1, Wp1, bp1, Wp2, bp2, Wp3, bp3):
    n, d = x.shape
    e = edge_index.shape[1]
    anf = additional_node_features
    amf = additional_message_features
    na = node_attr
    ea = edge_attr

    nch = 5
    ec = e // nch
    ch = ec // (_NW * _K)
    src4 = edge_index[0].reshape(nch, _NW, ch, _K)
    dst4 = edge_index[1].reshape(nch, _NW, ch, _K)
    amf4 = amf.reshape(nch, ec // 128, 1, 128)
    ea4 = ea.reshape(nch, ec // 128, 1, 128)
    zeros = jnp.zeros((n, d), jnp.float32)

    def split_m1(W):
        w = W[:, 0, :]
        return w[:d], w[d:d + 1], w[d + 1:2 * d + 1], w[2 * d + 1:2 * d + 2], \
            w[2 * d + 2:2 * d + 3]

    def split_u1(W):
        w = W[:, 0, :]
        return w[:d], w[d:d + 1], w[d + 1:d + 1 + d]

    rb = lambda b: b.reshape(1, d)
    wem = W_emb[:d, 0, :]
    wea = W_emb[d:d + 1, 0, :]
    se = 1.0 / math.sqrt(W_emb.shape[0])
    s1 = 1.0 / math.sqrt(Wm1_0.shape[0])
    s2 = 1.0 / math.sqrt(Wm2_0.shape[0])
    su1 = 1.0 / math.sqrt(Wu1_0.shape[0])
    su2 = 1.0 / math.sqrt(Wu2_0.shape[0])
    sp = 1.0 / math.sqrt(Wp1.shape[0])

    w1dm_0, w1da_0, w1sm_0, w1sa_0, w1amf_0 = split_m1(Wm1_0)
    w1dm_1, w1da_1, w1sm_1, w1sa_1, w1amf_1 = split_m1(Wm1_1)
    wu1m_0, wu1a_0, wu1g_0 = split_u1(Wu1_0)
    wu1m_1, wu1a_1, wu1g_1 = split_u1(Wu1_1)

    h0, ad0, as0 = _emb_call(x, anf, na, wem, wea, rb(b_emb),
                             w1dm_0, w1da_0, w1sm_0, w1sa_0, se)

    def layer(ad, as_, w1amf, bm1, wm2, bm2):
        parts = []
        for c in range(nch):
            td, ts = _sc_gather(ad, as_, dst4[c], src4[c])
            m2 = _edge_call(td, ts, amf4[c], ea4[c], w1amf, rb(bm1),
                            wm2[:, 0, :], rb(bm2), s1, s2)
            p = _sc_scatter(m2, dst4[c], zeros)
            parts += [p[0], p[1]]
        return parts

    parts0 = layer(ad0, as0, w1amf_0, bm1_0, Wm2_0, bm2_0)
    h1, ad1, as1 = _update_call(
        functools.partial(_update_prep_body, nparts=len(parts0),
                          su1=su1, su2=su2),
        [w1dm_1, w1da_1, w1sm_1, w1sa_1], [d, d, d],
        h0, anf, na, parts0,
        wu1m_0, wu1a_0, wu1g_0, rb(bu1_0), Wu2_0[:, 0, :], rb(bu2_0))

    parts1 = layer(ad1, as1, w1amf_1, bm1_1, Wm2_1, bm2_1)
    out = _update_call(
        functools.partial(_update_post_body, nparts=len(parts1),
                          su1=su1, su2=su2, sp=sp),
        [Wp1[:, 0, :], rb(bp1), Wp2[:, 0, :], rb(bp2), Wp3[:, 0, :], rb(bp3)],
        [d],
        h1, anf, na, parts1,
        wu1m_1, wu1a_1, wu1g_1, rb(bu1_1), Wu2_1[:, 0, :], rb(bu2_1))

    return out[0]


# speedup vs baseline: 4.5952x; 1.0858x over previous
import functools
import math

import jax
import jax.numpy as jnp
from jax import lax
from jax.experimental import pallas as pl
from jax.experimental.pallas import tpu as pltpu
from jax.experimental.pallas import tpu_sc as plsc

_NC = 2
_NS = 16
_NW = _NC * _NS
_K = 80
_GRP = 8

_RE = 1280
_RN = 1000


def _silu(v):
    return v * lax.logistic(v)


def _sc_gather(ad, as_, dst3, src3):
    n, d = ad.shape
    ch = dst3.shape[1]
    ew = ch * _K
    e = _NW * ew
    mesh = plsc.VectorSubcoreMesh(core_axis_name="c", subcore_axis_name="s")
    out_t = jax.ShapeDtypeStruct((e, d), jnp.float32)

    @functools.partial(
        pl.kernel,
        out_type=(out_t, out_t),
        mesh=mesh,
        scratch_types=[
            pltpu.VMEM((ch, _K), jnp.int32),
            pltpu.VMEM((ch, _K), jnp.int32),
            pltpu.VMEM((_GRP, _K, d), jnp.float32),
            pltpu.SemaphoreType.DMA,
            pltpu.SemaphoreType.DMA,
        ],
    )
    def gather_kernel(ad_hbm, as_hbm, dst_hbm, src_hbm, td_hbm, ts_hbm,
                      idxd_v, idxs_v, bufs, gsem, wsem):
        wid = lax.axis_index("s") * _NC + lax.axis_index("c")
        base = wid * ew
        pltpu.sync_copy(dst_hbm.at[wid], idxd_v)
        pltpu.sync_copy(src_hbm.at[wid], idxs_v)

        for table, idx_v, out in ((ad_hbm, idxd_v, td_hbm),
                                  (as_hbm, idxs_v, ts_hbm)):
            def group(g, nb, table=table, idx_v=idx_v, out=out):
                cps = [pltpu.async_copy(table.at[idx_v.at[g + b]],
                                        bufs.at[b], gsem)
                       for b in range(nb)]
                for cp in cps:
                    cp.wait()
                cps = [pltpu.async_copy(bufs.at[b],
                                        out.at[pl.ds(base + (g + b) * _K, _K)],
                                        wsem)
                       for b in range(nb)]
                for cp in cps:
                    cp.wait()

            nfull, rem = ch // _GRP, ch % _GRP

            @pl.loop(0, nfull)
            def _(i):
                group(i * _GRP, _GRP)

            if rem:
                group(nfull * _GRP, rem)

    return gather_kernel(ad, as_, dst3, src3)


def _sc_scatter(m2s, dst3s, zeros):
    ng = len(m2s)
    e, d = m2s[0].shape
    n = zeros.shape[0]
    ch = dst3s[0].shape[1]
    ew = ch * _K
    nr = (n // _NS) & ~7
    tail = n - nr * _NS
    mesh = plsc.VectorSubcoreMesh(core_axis_name="c", subcore_axis_name="s")

    grp = 4

    @functools.partial(
        pl.kernel,
        out_type=jax.ShapeDtypeStruct((_NC, n, d), jnp.float32),
        mesh=mesh,
        scratch_types=[
            pltpu.VMEM_SHARED((n, d), jnp.float32),
            pltpu.VMEM((ch, _K), jnp.int32),
            pltpu.VMEM((grp, _K, d), jnp.float32),
            pltpu.SemaphoreType.DMA,
            pltpu.SemaphoreType.DMA,
        ],
    )
    def scatter_kernel(*refs):
        m2_hbms = refs[:ng]
        dst_hbms = refs[ng:2 * ng]
        out_hbm = refs[2 * ng + 1]
        z_hbm = refs[2 * ng]
        agg_sh, idx_v, bufs, rsem, asem = refs[2 * ng + 2:]
        cid = lax.axis_index("c")
        sid = lax.axis_index("s")
        wid = sid * _NC + cid
        base = wid * ew
        pltpu.sync_copy(z_hbm.at[pl.ds(sid * nr, nr)],
                        agg_sh.at[pl.ds(sid * nr, nr)])
        if tail:
            @pl.when(sid == 0)
            def _():
                pltpu.sync_copy(z_hbm.at[pl.ds(nr * _NS, tail)],
                                agg_sh.at[pl.ds(nr * _NS, tail)])
        plsc.subcore_barrier()

        for m2_hbm, dst_hbm in zip(m2_hbms, dst_hbms):
            pltpu.sync_copy(dst_hbm.at[wid], idx_v)

            def group(g, nb, m2_hbm=m2_hbm):
                cps = [pltpu.async_copy(
                          m2_hbm.at[pl.ds(base + (g + b) * _K, _K)],
                          bufs.at[b], rsem)
                       for b in range(nb)]
                for cp in cps:
                    cp.wait()
                cps = [pltpu.async_copy(bufs.at[b],
                                        agg_sh.at[idx_v.at[g + b]],
                                        asem, add=True)
                       for b in range(nb)]
                for cp in cps:
                    cp.wait()

            nfull, rem = ch // grp, ch % grp

            @pl.loop(0, nfull)
            def _(i):
                group(i * grp, grp)

            if rem:
                group(nfull * grp, rem)

        plsc.subcore_barrier()
        pltpu.sync_copy(agg_sh.at[pl.ds(sid * nr, nr)],
                        out_hbm.at[cid, pl.ds(sid * nr, nr)])
        if tail:
            @pl.when(sid == 0)
            def _():
                pltpu.sync_copy(agg_sh.at[pl.ds(nr * _NS, tail)],
                                out_hbm.at[cid, pl.ds(nr * _NS, tail)])

    return scatter_kernel(*m2s, *dst3s, zeros)


def _dot(a, b):
    return jnp.dot(a, b, preferred_element_type=jnp.float32)


def _expand_rows(v, d):
    g = v.shape[0]
    ones = jnp.ones((1, d), jnp.float32)
    cols = [lax.dot_general(v[i:i + 1, :], ones, (((0,), (0,)), ((), ())),
                            preferred_element_type=jnp.float32)
            for i in range(g)]
    return jnp.concatenate(cols, axis=0)


def _edge_body(td_ref, ts_ref, amf_ref, ea_ref, w1a_ref, bm1_ref,
               w2_ref, bm2_ref, out_ref, *, s1, s2):
    d = td_ref.shape[1]
    gr = amf_ref.shape[0]
    amf = _expand_rows(amf_ref[...].reshape(gr, 128), d)
    ea = _expand_rows(ea_ref[...].reshape(gr, 128), d)
    t = td_ref[...] + ts_ref[...]
    pre = (t + amf * w1a_ref[...]) * (ea * s1) + bm1_ref[...]
    m1 = _silu(pre)
    pre2 = _dot(m1, w2_ref[...]) * (ea * s2) + bm2_ref[...]
    out_ref[...] = _silu(pre2)


def _edge_call(td, ts, amf_l, ea_l, w1a, bm1, w2, bm2, s1, s2):
    e, d = td.shape
    grid = e // _RE
    gr = _RE // 128
    row = lambda i: (i, 0)
    zero = lambda i: (0, 0)
    return pl.pallas_call(
        functools.partial(_edge_body, s1=s1, s2=s2),
        grid=(grid,),
        in_specs=[
            pl.BlockSpec((_RE, d), row),
            pl.BlockSpec((_RE, d), row),
            pl.BlockSpec((gr, 1, 128), lambda i: (i, 0, 0)),
            pl.BlockSpec((gr, 1, 128), lambda i: (i, 0, 0)),
            pl.BlockSpec((1, d), zero),
            pl.BlockSpec((1, d), zero),
            pl.BlockSpec((d, d), zero),
            pl.BlockSpec((1, d), zero),
        ],
        out_specs=pl.BlockSpec((_RE, d), row),
        out_shape=jax.ShapeDtypeStruct((e, d), jnp.float32),
    )(td, ts, amf_l, ea_l, w1a, bm1, w2, bm2)


def _emb_body(x_ref, anf_ref, na_ref, wem_ref, wea_ref, be_ref,
              w1dm_ref, w1da_ref, w1sm_ref, w1sa_ref,
              h_ref, ad_ref, as_ref, *, se):
    anf = anf_ref[...]
    na = na_ref[...]
    h = (_dot(x_ref[...], wem_ref[...]) + anf * wea_ref[...]) * (na * se) \
        + be_ref[...]
    h_ref[...] = h
    ad_ref[...] = _dot(h, w1dm_ref[...]) + anf * w1da_ref[...]
    as_ref[...] = _dot(h, w1sm_ref[...]) + anf * w1sa_ref[...]


def _emb_call(x, anf, na, wem, wea, be, w1dm, w1da, w1sm, w1sa, se):
    n, d = x.shape
    grid = n // _RN
    row = lambda i: (i, 0)
    zero = lambda i: (0, 0)
    nd = jax.ShapeDtypeStruct((n, d), jnp.float32)
    return pl.pallas_call(
        functools.partial(_emb_body, se=se),
        grid=(grid,),
        in_specs=[
            pl.BlockSpec((_RN, d), row),
            pl.BlockSpec((_RN, 1), row),
            pl.BlockSpec((_RN, 1), row),
            pl.BlockSpec((d, d), zero),
            pl.BlockSpec((1, d), zero),
            pl.BlockSpec((1, d), zero),
            pl.BlockSpec((d, d), zero),
            pl.BlockSpec((1, d), zero),
            pl.BlockSpec((d, d), zero),
            pl.BlockSpec((1, d), zero),
        ],
        out_specs=[pl.BlockSpec((_RN, d), row)] * 3,
        out_shape=[nd, nd, nd],
    )(x, anf, na, wem, wea, be, w1dm, w1da, w1sm, w1sa)


def _update_common(h_ref, anf_ref, na_ref, parts,
                   wu1m_ref, wu1a_ref, wu1g_ref, bu1_ref, wu2_ref, bu2_ref,
                   su1, su2):
    anf = anf_ref[...]
    na = na_ref[...]
    h = h_ref[...]
    agg = parts[0][...]
    for p in parts[1:]:
        agg = agg + p[...]
    pre = (_dot(h, wu1m_ref[...]) + anf * wu1a_ref[...]
           + _dot(agg, wu1g_ref[...])) * (na * su1) + bu1_ref[...]
    u = _silu(pre)
    u2 = _dot(u, wu2_ref[...]) * (na * su2) + bu2_ref[...]
    return h + u2, anf, na


def _update_prep_body(h_ref, anf_ref, na_ref, *rest, nparts, su1, su2):
    parts = rest[:nparts]
    (wu1m_ref, wu1a_ref, wu1g_ref, bu1_ref, wu2_ref, bu2_ref,
     w1dm_ref, w1da_ref, w1sm_ref, w1sa_ref,
     hn_ref, ad_ref, as_ref) = rest[nparts:]
    hn, anf, _ = _update_common(h_ref, anf_ref, na_ref, parts,
                                wu1m_ref, wu1a_ref, wu1g_ref, bu1_ref,
                                wu2_ref, bu2_ref, su1, su2)
    hn_ref[...] = hn
    ad_ref[...] = _dot(hn, w1dm_ref[...]) + anf * w1da_ref[...]
    as_ref[...] = _dot(hn, w1sm_ref[...]) + anf * w1sa_ref[...]


def _update_post_body(h_ref, anf_ref, na_ref, *rest, nparts, su1, su2, sp):
    parts = rest[:nparts]
    (wu1m_ref, wu1a_ref, wu1g_ref, bu1_ref, wu2_ref, bu2_ref,
     wp1_ref, bp1_ref, wp2_ref, bp2_ref, wp3_ref, bp3_ref,
     out_ref) = rest[nparts:]
    hn, _, na = _update_common(h_ref, anf_ref, na_ref, parts,
                               wu1m_ref, wu1a_ref, wu1g_ref, bu1_ref,
                               wu2_ref, bu2_ref, su1, su2)
    q1 = _silu(_dot(hn, wp1_ref[...]) * (na * sp) + bp1_ref[...])
    q2 = _dot(q1, wp2_ref[...]) * (na * sp) + bp2_ref[...]
    out_ref[...] = _dot(q2, wp3_ref[...]) * (na * sp) + bp3_ref[...]


def _update_call(body, extra_w, out_widths, h, anf, na, parts,
                 wu1m, wu1a, wu1g, bu1, wu2, bu2):
    n, d = h.shape
    grid = n // _RN
    row = lambda i: (i, 0)
    zero = lambda i: (0, 0)
    extra_specs = []
    for w in extra_w:
        extra_specs.append(pl.BlockSpec(w.shape, zero))
    return pl.pallas_call(
        body,
        grid=(grid,),
        in_specs=[
            pl.BlockSpec((_RN, d), row),
            pl.BlockSpec((_RN, 1), row),
            pl.BlockSpec((_RN, 1), row),
        ] + [pl.BlockSpec((_RN, d), row)] * len(parts) + [
            pl.BlockSpec((d, d), zero),
            pl.BlockSpec((1, d), zero),
            pl.BlockSpec((d, d), zero),
            pl.BlockSpec((1, d), zero),
            pl.BlockSpec((d, d), zero),
            pl.BlockSpec((1, d), zero),
        ] + extra_specs,
        out_specs=[pl.BlockSpec((_RN, w), row) for w in out_widths],
        out_shape=[jax.ShapeDtypeStruct((n, w), jnp.float32)
                   for w in out_widths],
    )(h, anf, na, *parts, wu1m, wu1a, wu1g, bu1, wu2, bu2, *extra_w)


def kernel(x, edge_index, edge_attr, node_attr, additional_message_features,
           additional_node_features, W_emb, b_emb, Wm1_0, bm1_0, Wm2_0, bm2_0,
           Wu1_0, bu1_0, Wu2_0, bu2_0, Wm1_1, bm1_1, Wm2_1, bm2_1,
           Wu1_1, bu1_1, Wu2_1, bu2_1, Wp1, bp1, Wp2, bp2, Wp3, bp3):
    n, d = x.shape
    e = edge_index.shape[1]
    anf = additional_node_features
    amf = additional_message_features
    na = node_attr
    ea = edge_attr

    nch = 5
    ec = e // nch
    ch = ec // (_NW * _K)
    src4 = edge_index[0].reshape(nch, _NW, ch, _K)
    dst4 = edge_index[1].reshape(nch, _NW, ch, _K)
    amf4 = amf.reshape(nch, ec // 128, 1, 128)
    ea4 = ea.reshape(nch, ec // 128, 1, 128)
    zeros = jnp.zeros((n, d), jnp.float32)

    def split_m1(W):
        w = W[:, 0, :]
        return w[:d], w[d:d + 1], w[d + 1:2 * d + 1], w[2 * d + 1:2 * d + 2], \
            w[2 * d + 2:2 * d + 3]

    def split_u1(W):
        w = W[:, 0, :]
        return w[:d], w[d:d + 1], w[d + 1:d + 1 + d]

    rb = lambda b: b.reshape(1, d)
    wem = W_emb[:d, 0, :]
    wea = W_emb[d:d + 1, 0, :]
    se = 1.0 / math.sqrt(W_emb.shape[0])
    s1 = 1.0 / math.sqrt(Wm1_0.shape[0])
    s2 = 1.0 / math.sqrt(Wm2_0.shape[0])
    su1 = 1.0 / math.sqrt(Wu1_0.shape[0])
    su2 = 1.0 / math.sqrt(Wu2_0.shape[0])
    sp = 1.0 / math.sqrt(Wp1.shape[0])

    w1dm_0, w1da_0, w1sm_0, w1sa_0, w1amf_0 = split_m1(Wm1_0)
    w1dm_1, w1da_1, w1sm_1, w1sa_1, w1amf_1 = split_m1(Wm1_1)
    wu1m_0, wu1a_0, wu1g_0 = split_u1(Wu1_0)
    wu1m_1, wu1a_1, wu1g_1 = split_u1(Wu1_1)

    h0, ad0, as0 = _emb_call(x, anf, na, wem, wea, rb(b_emb),
                             w1dm_0, w1da_0, w1sm_0, w1sa_0, se)

    def layer(ad, as_, w1amf, bm1, wm2, bm2):
        m2s = []
        for c in range(nch):
            td, ts = _sc_gather(ad, as_, dst4[c], src4[c])
            m2s.append(_edge_call(td, ts, amf4[c], ea4[c], w1amf, rb(bm1),
                                  wm2[:, 0, :], rb(bm2), s1, s2))
        parts = []
        for lo, hi in ((0, 3), (3, nch)):
            p = _sc_scatter(m2s[lo:hi], [dst4[c] for c in range(lo, hi)],
                            zeros)
            parts += [p[0], p[1]]
        return parts

    parts0 = layer(ad0, as0, w1amf_0, bm1_0, Wm2_0, bm2_0)
    h1, ad1, as1 = _update_call(
        functools.partial(_update_prep_body, nparts=len(parts0),
                          su1=su1, su2=su2),
        [w1dm_1, w1da_1, w1sm_1, w1sa_1], [d, d, d],
        h0, anf, na, parts0,
        wu1m_0, wu1a_0, wu1g_0, rb(bu1_0), Wu2_0[:, 0, :], rb(bu2_0))

    parts1 = layer(ad1, as1, w1amf_1, bm1_1, Wm2_1, bm2_1)
    out = _update_call(
        functools.partial(_update_post_body, nparts=len(parts1),
                          su1=su1, su2=su2, sp=sp),
        [Wp1[:, 0, :], rb(bp1), Wp2[:, 0, :], rb(bp2), Wp3[:, 0, :], rb(bp3)],
        [d],
        h1, anf, na, parts1,
        wu1m_1, wu1a_1, wu1g_1, rb(bu1_1), Wu2_1[:, 0, :], rb(bu2_1))

    return out[0]
